# baseline (speedup 1.0000x reference)
import dataclasses
import functools

import jax
import jax.numpy as jnp
from jax import lax
from jax.experimental import pallas as pl
from jax.experimental.pallas import tpu as pltpu
from jax.experimental.pallas import tpu_sc as plsc

N_NODES = 10000
N_EDGES = 320000
D = 128
DH = D

NC = 2
NS = 16
LANES = 16

EPS = N_EDGES // (NC * NS)
K = 16
NCHUNK = EPS // K
NBUF = 5

ROWS_PER_TILE = 624
TAIL_ROW0 = NS * ROWS_PER_TILE
TAIL_ROWS = N_NODES - TAIL_ROW0


def _sc_aggregate(x_halves, src, dst, ew, zeros):
    mesh = plsc.VectorSubcoreMesh(core_axis_name="c", subcore_axis_name="s")
    cp = pltpu.CompilerParams()
    if "needs_layout_passes" in pltpu.CompilerParams.__dataclass_fields__:
        cp = dataclasses.replace(cp, needs_layout_passes=False)

    @functools.partial(
        pl.kernel,
        compiler_params=cp,
        out_type=jax.ShapeDtypeStruct((NC, N_NODES, DH), jnp.float32),
        mesh=mesh,
        scratch_types=[
            pltpu.VMEM((NBUF, K), jnp.int32),
            pltpu.VMEM((NBUF, K), jnp.int32),
            pltpu.VMEM((NBUF, K), jnp.float32),
            pltpu.VMEM((NBUF, K, DH), jnp.float32),
            pltpu.VMEM_SHARED((N_NODES, DH), jnp.float32),
        ]
        + [pltpu.SemaphoreType.DMA] * (5 * NBUF),
    )
    def agg_kernel(x_hbm, src_hbm, dst_hbm, ew_hbm, zeros_hbm, out_hbm,
                   src_v, dst_v, ew_v, rows_v, acc, *sems):
        gsem = sems[0 * NBUF:1 * NBUF]
        ssem = sems[1 * NBUF:2 * NBUF]
        srcsem = sems[2 * NBUF:3 * NBUF]
        dstsem = sems[3 * NBUF:4 * NBUF]
        ewsem = sems[4 * NBUF:5 * NBUF]
        c = lax.axis_index("c")
        s = lax.axis_index("s")
        wid = c * NS + s

        for b in range(NBUF):
            pltpu.async_copy(src_hbm.at[wid].at[b], src_v.at[b], srcsem[b])
            pltpu.async_copy(ew_hbm.at[wid].at[b], ew_v.at[b], ewsem[b])

        row0 = s * ROWS_PER_TILE
        pltpu.sync_copy(zeros_hbm.at[pl.ds(row0, ROWS_PER_TILE)],
                        acc.at[pl.ds(row0, ROWS_PER_TILE)])

        @pl.when(s == 0)
        def _zero_tail():
            pltpu.sync_copy(zeros_hbm.at[pl.ds(TAIL_ROW0, TAIL_ROWS)],
                            acc.at[pl.ds(TAIL_ROW0, TAIL_ROWS)])

        plsc.subcore_barrier()

        @pl.loop(0, NCHUNK, step=NBUF)
        def _super(ci):
            gathers = []
            for b in range(NBUF):
                @pl.when(ci > 0)
                def _wait_scatter(b=b):
                    pltpu.make_async_copy(x_hbm.at[pl.ds(0, K)],
                                          rows_v.at[b], ssem[b]).wait()

                pltpu.async_copy(dst_hbm.at[wid].at[ci + b], dst_v.at[b],
                                 dstsem[b])
                pltpu.make_async_copy(src_hbm.at[0].at[0], src_v.at[b],
                                      srcsem[b]).wait()
                gathers.append(
                    pltpu.async_copy(x_hbm.at[src_v.at[b]], rows_v.at[b],
                                     gsem[b]))

            for b in range(NBUF):
                gathers[b].wait()

                @pl.when(ci + b + NBUF < NCHUNK)
                def _prefetch_src(b=b):
                    pltpu.async_copy(src_hbm.at[wid].at[ci + b + NBUF],
                                     src_v.at[b], srcsem[b])

                pltpu.make_async_copy(ew_hbm.at[0].at[0], ew_v.at[b],
                                      ewsem[b]).wait()
                rb = rows_v.at[b]

                @plsc.parallel_loop(0, K, unroll=8)
                def _edge(i, b=b, rb=rb):
                    bidx = jnp.full((LANES,), b, jnp.int32)
                    widx = jnp.full((LANES,), 0, jnp.int32) + i
                    w = plsc.load_gather(ew_v, [bidx, widx])
                    for j in range(DH // LANES):
                        sl = pl.ds(j * LANES, LANES)
                        rb[i, sl] = rb[i, sl] * w

                @pl.when(ci + b + NBUF < NCHUNK)
                def _prefetch_ew(b=b):
                    pltpu.async_copy(ew_hbm.at[wid].at[ci + b + NBUF],
                                     ew_v.at[b], ewsem[b])

                pltpu.make_async_copy(dst_hbm.at[0].at[0], dst_v.at[b],
                                      dstsem[b]).wait()
                pltpu.async_copy(rb, acc.at[dst_v.at[b]], ssem[b], add=True)

        for b in range(NBUF):
            pltpu.make_async_copy(x_hbm.at[pl.ds(0, K)], rows_v.at[b],
                                  ssem[b]).wait()

        plsc.subcore_barrier()
        pltpu.sync_copy(acc.at[pl.ds(row0, ROWS_PER_TILE)],
                        out_hbm.at[c].at[pl.ds(row0, ROWS_PER_TILE)])

        @pl.when(s == 0)
        def _copy_tail():
            pltpu.sync_copy(acc.at[pl.ds(TAIL_ROW0, TAIL_ROWS)],
                            out_hbm.at[c].at[pl.ds(TAIL_ROW0, TAIL_ROWS)])

    return agg_kernel(x_halves, src, dst, ew, zeros)


def _tc_transform(partials, W):
    BM = 400

    def body(p_ref, w_ref, o_ref):
        o_ref[...] = jnp.dot(p_ref[0] + p_ref[1], w_ref[...],
                             preferred_element_type=jnp.float32,
                             precision=lax.Precision.HIGHEST)

    return pl.pallas_call(
        body,
        grid=(N_NODES // BM,),
        in_specs=[
            pl.BlockSpec((NC, BM, DH), lambda i: (0, i, 0)),
            pl.BlockSpec((D, D), lambda i: (0, 0)),
        ],
        out_specs=pl.BlockSpec((BM, D), lambda i: (i, 0)),
        out_shape=jax.ShapeDtypeStruct((N_NODES, D), jnp.float32),
    )(partials, W)


def kernel(x, edge_index, edge_weight, W):
    NW = NC * NS
    x32 = x.astype(jnp.float32)
    src = edge_index[0].astype(jnp.int32).reshape(NW, NCHUNK, K)
    dst = edge_index[1].astype(jnp.int32).reshape(NW, NCHUNK, K)
    ew = edge_weight.astype(jnp.float32).reshape(NW, NCHUNK, K)
    zeros = jnp.zeros((N_NODES, DH), jnp.float32)
    partials = _sc_aggregate(x32, src, dst, ew, zeros)
    return _tc_transform(partials, W)


# speedup vs baseline: 1.4218x; 1.4218x over previous
import dataclasses
import functools

import jax
import jax.numpy as jnp
from jax import lax
from jax.experimental import pallas as pl
from jax.experimental.pallas import tpu as pltpu
from jax.experimental.pallas import tpu_sc as plsc

N_NODES = 10000
N_EDGES = 320000
D = 128
DH = D

NC = 2
NS = 16
LANES = 16

EPS = N_EDGES // (NC * NS)
K = 40
NCHUNK = EPS // K
NBUF = 5

ROWS_PER_TILE = 624
TAIL_ROW0 = NS * ROWS_PER_TILE
TAIL_ROWS = N_NODES - TAIL_ROW0


def _sc_aggregate(x_halves, src, dst, ew, zeros):
    mesh = plsc.VectorSubcoreMesh(core_axis_name="c", subcore_axis_name="s")
    cp = pltpu.CompilerParams()
    if "needs_layout_passes" in pltpu.CompilerParams.__dataclass_fields__:
        cp = dataclasses.replace(cp, needs_layout_passes=False)

    @functools.partial(
        pl.kernel,
        compiler_params=cp,
        out_type=jax.ShapeDtypeStruct((NC, N_NODES, DH), jnp.float32),
        mesh=mesh,
        scratch_types=[
            pltpu.VMEM((NBUF, K), jnp.int32),
            pltpu.VMEM((NBUF, K), jnp.int32),
            pltpu.VMEM((NBUF, K), jnp.float32),
            pltpu.VMEM((NBUF, K, DH), jnp.float32),
            pltpu.VMEM_SHARED((N_NODES, DH), jnp.float32),
        ]
        + [pltpu.SemaphoreType.DMA] * (5 * NBUF),
    )
    def agg_kernel(x_hbm, src_hbm, dst_hbm, ew_hbm, zeros_hbm, out_hbm,
                   src_v, dst_v, ew_v, rows_v, acc, *sems):
        gsem = sems[0 * NBUF:1 * NBUF]
        ssem = sems[1 * NBUF:2 * NBUF]
        srcsem = sems[2 * NBUF:3 * NBUF]
        dstsem = sems[3 * NBUF:4 * NBUF]
        ewsem = sems[4 * NBUF:5 * NBUF]
        c = lax.axis_index("c")
        s = lax.axis_index("s")
        wid = c * NS + s

        for b in range(NBUF):
            pltpu.async_copy(src_hbm.at[wid].at[b], src_v.at[b], srcsem[b])
            pltpu.async_copy(ew_hbm.at[wid].at[b], ew_v.at[b], ewsem[b])

        row0 = s * ROWS_PER_TILE
        pltpu.sync_copy(zeros_hbm.at[pl.ds(row0, ROWS_PER_TILE)],
                        acc.at[pl.ds(row0, ROWS_PER_TILE)])

        @pl.when(s == 0)
        def _zero_tail():
            pltpu.sync_copy(zeros_hbm.at[pl.ds(TAIL_ROW0, TAIL_ROWS)],
                            acc.at[pl.ds(TAIL_ROW0, TAIL_ROWS)])

        plsc.subcore_barrier()

        @pl.loop(0, NCHUNK, step=NBUF)
        def _super(ci):
            gathers = []
            for b in range(NBUF):
                @pl.when(ci > 0)
                def _wait_scatter(b=b):
                    pltpu.make_async_copy(x_hbm.at[pl.ds(0, K)],
                                          rows_v.at[b], ssem[b]).wait()

                pltpu.async_copy(dst_hbm.at[wid].at[ci + b], dst_v.at[b],
                                 dstsem[b])
                pltpu.make_async_copy(src_hbm.at[0].at[0], src_v.at[b],
                                      srcsem[b]).wait()
                gathers.append(
                    pltpu.async_copy(x_hbm.at[src_v.at[b]], rows_v.at[b],
                                     gsem[b]))

            for b in range(NBUF):
                gathers[b].wait()

                @pl.when(ci + b + NBUF < NCHUNK)
                def _prefetch_src(b=b):
                    pltpu.async_copy(src_hbm.at[wid].at[ci + b + NBUF],
                                     src_v.at[b], srcsem[b])

                pltpu.make_async_copy(ew_hbm.at[0].at[0], ew_v.at[b],
                                      ewsem[b]).wait()
                rb = rows_v.at[b]

                @plsc.parallel_loop(0, K, unroll=8)
                def _edge(i, b=b, rb=rb):
                    bidx = jnp.full((LANES,), b, jnp.int32)
                    widx = jnp.full((LANES,), 0, jnp.int32) + i
                    w = plsc.load_gather(ew_v, [bidx, widx])
                    for j in range(DH // LANES):
                        sl = pl.ds(j * LANES, LANES)
                        rb[i, sl] = rb[i, sl] * w

                @pl.when(ci + b + NBUF < NCHUNK)
                def _prefetch_ew(b=b):
                    pltpu.async_copy(ew_hbm.at[wid].at[ci + b + NBUF],
                                     ew_v.at[b], ewsem[b])

                pltpu.make_async_copy(dst_hbm.at[0].at[0], dst_v.at[b],
                                      dstsem[b]).wait()
                pltpu.async_copy(rb, acc.at[dst_v.at[b]], ssem[b], add=True)

        for b in range(NBUF):
            pltpu.make_async_copy(x_hbm.at[pl.ds(0, K)], rows_v.at[b],
                                  ssem[b]).wait()

        plsc.subcore_barrier()
        pltpu.sync_copy(acc.at[pl.ds(row0, ROWS_PER_TILE)],
                        out_hbm.at[c].at[pl.ds(row0, ROWS_PER_TILE)])

        @pl.when(s == 0)
        def _copy_tail():
            pltpu.sync_copy(acc.at[pl.ds(TAIL_ROW0, TAIL_ROWS)],
                            out_hbm.at[c].at[pl.ds(TAIL_ROW0, TAIL_ROWS)])

    return agg_kernel(x_halves, src, dst, ew, zeros)


def _tc_transform(partials, W):
    BM = 400

    def body(p_ref, w_ref, o_ref):
        o_ref[...] = jnp.dot(p_ref[0] + p_ref[1], w_ref[...],
                             preferred_element_type=jnp.float32,
                             precision=lax.Precision.HIGHEST)

    return pl.pallas_call(
        body,
        grid=(N_NODES // BM,),
        in_specs=[
            pl.BlockSpec((NC, BM, DH), lambda i: (0, i, 0)),
            pl.BlockSpec((D, D), lambda i: (0, 0)),
        ],
        out_specs=pl.BlockSpec((BM, D), lambda i: (i, 0)),
        out_shape=jax.ShapeDtypeStruct((N_NODES, D), jnp.float32),
    )(partials, W)


def kernel(x, edge_index, edge_weight, W):
    NW = NC * NS
    x32 = x.astype(jnp.float32)
    src = edge_index[0].astype(jnp.int32).reshape(NW, NCHUNK, K)
    dst = edge_index[1].astype(jnp.int32).reshape(NW, NCHUNK, K)
    ew = edge_weight.astype(jnp.float32).reshape(NW, NCHUNK, K)
    zeros = jnp.zeros((N_NODES, DH), jnp.float32)
    partials = _sc_aggregate(x32, src, dst, ew, zeros)
    return _tc_transform(partials, W)
